# Initial kernel scaffold; baseline (speedup 1.0000x reference)
import jax
import jax.numpy as jnp
from jax.experimental import pallas as pl


def kernel(x, edge_index, W1, b1, W2, b2, W3, b3, We, be, Wd1, bd1, Wd2, bd2):
    raise NotImplementedError("write your pallas kernel here")


# speedup vs baseline: 2.2716x; 2.2716x over previous
import functools

import jax
import jax.numpy as jnp
from jax import lax
from jax.experimental import pallas as pl
from jax.experimental.pallas import tpu as pltpu
from jax.experimental.pallas import tpu_sc as plsc

N = 10000
E = 160000
IN = 256
H = 1024
LAT = 256

MP = 10240
SHIFT = 14
SMASK = (1 << SHIFT) - 1
BM = 512
BN = 512

_NC, _NS = 2, 16


def _dinv_block(lo_ref, hi_ref):
    deg = (hi_ref[...] - lo_ref[...] + 1).astype(jnp.float32)
    return lax.rsqrt(deg)


def _row_mask(i, blk):
    rows = i * BM + lax.broadcasted_iota(jnp.int32, (BM, 1), 0)
    return jnp.where(rows < N, blk, 0.0)


def _mm_first_body(x_ref, w_ref, lo_ref, hi_ref, o_ref):
    acc = jnp.dot(x_ref[...], w_ref[...], preferred_element_type=jnp.float32)
    o_ref[...] = acc * _dinv_block(lo_ref, hi_ref)


def _mm_fused_body(raw_ref, gs_ref, b_ref, w_ref, lo_ref, hi_ref, o_ref):
    dinv = _dinv_block(lo_ref, hi_ref)
    a = jax.nn.relu(dinv * (raw_ref[...] + gs_ref[...]) + b_ref[...])
    a = _row_mask(pl.program_id(0), a)
    acc = jnp.dot(a, w_ref[...], preferred_element_type=jnp.float32)
    o_ref[...] = acc * dinv


def _mm_enc_body(raw_ref, gs_ref, b_ref, w_ref, be_ref, lo_ref, hi_ref,
                 z_ref, zg_ref):
    dinv = _dinv_block(lo_ref, hi_ref)
    a = jax.nn.relu(dinv * (raw_ref[...] + gs_ref[...]) + b_ref[...])
    i = pl.program_id(0)
    a = _row_mask(i, a)
    z = jax.nn.relu(
        jnp.dot(a, w_ref[...], preferred_element_type=jnp.float32) + be_ref[...])
    z_ref[...] = z

    @pl.when(i == 0)
    def _():
        zg_ref[...] = jnp.zeros_like(zg_ref)

    zg_ref[...] += jnp.sum(_row_mask(i, z), axis=0, keepdims=True)

    @pl.when(i == pl.num_programs(0) - 1)
    def _():
        zg_ref[...] *= (1.0 / N)


def _mm_bias_body(x_ref, w_ref, b_ref, o_ref, *, relu):
    acc = jnp.dot(x_ref[...], w_ref[...], preferred_element_type=jnp.float32)
    acc = acc + b_ref[...]
    o_ref[...] = jax.nn.relu(acc) if relu else acc


def _col(i, j):
    return (i, 0)


def _mm_first(x, w, lo, hi):
    k, n = w.shape
    return pl.pallas_call(
        _mm_first_body,
        grid=(MP // BM, n // BN),
        in_specs=[
            pl.BlockSpec((BM, k), lambda i, j: (i, 0)),
            pl.BlockSpec((k, BN), lambda i, j: (0, j)),
            pl.BlockSpec((BM, 1), _col),
            pl.BlockSpec((BM, 1), _col),
        ],
        out_specs=pl.BlockSpec((BM, BN), lambda i, j: (i, j)),
        out_shape=jax.ShapeDtypeStruct((MP, n), jnp.float32),
    )(x, w, lo, hi)


def _mm_fused(raw, gs, b, w, lo, hi):
    k, n = w.shape
    return pl.pallas_call(
        _mm_fused_body,
        grid=(MP // BM, n // BN),
        in_specs=[
            pl.BlockSpec((BM, k), lambda i, j: (i, 0)),
            pl.BlockSpec((BM, k), lambda i, j: (i, 0)),
            pl.BlockSpec((1, k), lambda i, j: (0, 0)),
            pl.BlockSpec((k, BN), lambda i, j: (0, j)),
            pl.BlockSpec((BM, 1), _col),
            pl.BlockSpec((BM, 1), _col),
        ],
        out_specs=pl.BlockSpec((BM, BN), lambda i, j: (i, j)),
        out_shape=jax.ShapeDtypeStruct((MP, n), jnp.float32),
    )(raw, gs, b, w, lo, hi)


def _mm_enc(raw, gs, b, w, be, lo, hi):
    k, n = w.shape
    return pl.pallas_call(
        _mm_enc_body,
        grid=(MP // BM, 1),
        in_specs=[
            pl.BlockSpec((BM, k), lambda i, j: (i, 0)),
            pl.BlockSpec((BM, k), lambda i, j: (i, 0)),
            pl.BlockSpec((1, k), lambda i, j: (0, 0)),
            pl.BlockSpec((k, n), lambda i, j: (0, 0)),
            pl.BlockSpec((1, n), lambda i, j: (0, 0)),
            pl.BlockSpec((BM, 1), _col),
            pl.BlockSpec((BM, 1), _col),
        ],
        out_specs=[
            pl.BlockSpec((BM, n), lambda i, j: (i, 0)),
            pl.BlockSpec((1, n), lambda i, j: (0, 0)),
        ],
        out_shape=[
            jax.ShapeDtypeStruct((MP, n), jnp.float32),
            jax.ShapeDtypeStruct((1, n), jnp.float32),
        ],
    )(raw, gs, b, w, be, lo, hi)


def _mm_bias(x, w, b, relu):
    k, n = w.shape
    bn = min(BN, n)
    return pl.pallas_call(
        functools.partial(_mm_bias_body, relu=relu),
        grid=(MP // BM, n // bn),
        in_specs=[
            pl.BlockSpec((BM, k), lambda i, j: (i, 0)),
            pl.BlockSpec((k, bn), lambda i, j: (0, j)),
            pl.BlockSpec((1, bn), lambda i, j: (0, j)),
        ],
        out_specs=pl.BlockSpec((BM, bn), lambda i, j: (i, j)),
        out_shape=jax.ShapeDtypeStruct((MP, n), jnp.float32),
    )(x, w, b)


R = 32
NGROUP = MP // R
NGPT = NGROUP // 32
ZROW = MP - 1
ESAFE = E

@functools.cache
def _make_sc_conv():
    mesh = plsc.VectorSubcoreMesh(core_axis_name="c", subcore_axis_name="s")

    @functools.partial(
        pl.kernel,
        out_type=jax.ShapeDtypeStruct((MP, H), jnp.float32),
        mesh=mesh,
        scratch_types=[
            pltpu.VMEM((R,), jnp.int32),
            pltpu.VMEM((R,), jnp.int32),
            pltpu.VMEM((R,), jnp.int32),
            pltpu.VMEM((R,), jnp.int32),
            pltpu.VMEM((R,), jnp.int32),
            pltpu.VMEM((16,), jnp.int32),
            pltpu.VMEM((R, H), jnp.float32),
            pltpu.VMEM((R, H), jnp.float32),
            pltpu.SemaphoreType.DMA,
        ],
    )
    def sc_conv(goffs_hbm, gdegs_hbm, rows_hbm, meta_hbm, ssrc_hbm, gs_hbm,
                raw_hbm, offv, degv, rowsb, eidxb, srcb, metav, acc, buf, sem):
        c = lax.axis_index("c")
        s = lax.axis_index("s")
        wid = s * _NC + c

        def build_eidx(t):
            for j in range(R // 16):
                sl = pl.ds(j * 16, 16)
                off16 = offv[sl]
                deg16 = degv[sl]
                eidxb[sl] = jnp.where(deg16 > t, off16 + t, ESAFE)

        def gather_rows(t, dst):
            build_eidx(t)
            pltpu.async_copy(ssrc_hbm.at[eidxb], srcb, sem).wait()
            pltpu.async_copy(gs_hbm.at[srcb], dst, sem).wait()

        def add_buf():
            def row_body(i, _):
                for k in range(H // 16):
                    sl = pl.ds(k * 16, 16)
                    plsc.addupdate(acc.at[i, sl], buf[i, sl])
                return 0
            lax.fori_loop(0, R, row_body, 0)

        for k in range(NGPT):
            g = wid + 32 * k
            gbase = pl.multiple_of(g * R, 8)
            pltpu.sync_copy(goffs_hbm.at[pl.ds(gbase, R)], offv)
            pltpu.sync_copy(gdegs_hbm.at[pl.ds(gbase, R)], degv)
            pltpu.sync_copy(rows_hbm.at[pl.ds(gbase, R)], rowsb)
            pltpu.sync_copy(meta_hbm.at[pl.ds(pl.multiple_of(g * 16, 8), 16)], metav)
            lmax = metav[...][0]
            gather_rows(0, acc)

            def layer(t, _):
                gather_rows(t, buf)
                add_buf()
                return 0

            lax.fori_loop(1, lmax, layer, 0)
            pltpu.async_copy(acc, raw_hbm.at[rowsb], sem).wait()

    return sc_conv


def _sc_conv(*args):
    return _make_sc_conv()(*args)


def kernel(x, edge_index, W1, b1, W2, b2, W3, b3, We, be, Wd1, bd1, Wd2, bd2):
    src = edge_index[0].astype(jnp.int32)
    dst = edge_index[1].astype(jnp.int32)

    key = jnp.bitwise_or(lax.shift_left(dst, SHIFT), src)
    skey = jnp.sort(key)
    ssrc = jnp.bitwise_and(skey, SMASK)
    sdst = lax.shift_right_logical(skey, SHIFT)
    ssrc_p = jnp.concatenate([ssrc, jnp.full((64,), ZROW, jnp.int32)])

    offs = jnp.searchsorted(
        sdst, jnp.arange(MP + 8, dtype=jnp.int32), side="left").astype(jnp.int32)
    lo = offs[:MP].reshape(MP, 1)
    hi = offs[1:MP + 1].reshape(MP, 1)
    deg = (offs[1:MP + 1] - offs[:MP])

    order = jnp.argsort(-deg).astype(jnp.int32)
    goffs = offs[:MP][order]
    gdegs = deg[order]
    lmax = jnp.maximum(jnp.max(gdegs.reshape(NGROUP, R), axis=1), 1)
    meta = jnp.concatenate(
        [lmax[:, None], jnp.zeros((NGROUP, 15), jnp.int32)], axis=1).reshape(-1)

    x_p = jnp.pad(x, ((0, MP - N), (0, 0)))
    b1r, b2r, b3r = b1.reshape(1, H), b2.reshape(1, H), b3.reshape(1, H)
    ber, bd1r, bd2r = be.reshape(1, LAT), bd1.reshape(1, H), bd2.reshape(1, IN)

    gs1 = _mm_first(x_p, W1, lo, hi)
    raw1 = _sc_conv(goffs, gdegs, order, meta, ssrc_p, gs1)
    gs2 = _mm_fused(raw1, gs1, b1r, W2, lo, hi)
    raw2 = _sc_conv(goffs, gdegs, order, meta, ssrc_p, gs2)
    gs3 = _mm_fused(raw2, gs2, b2r, W3, lo, hi)
    raw3 = _sc_conv(goffs, gdegs, order, meta, ssrc_p, gs3)
    z_p, zg = _mm_enc(raw3, gs3, b3r, We, ber, lo, hi)
    dd = _mm_bias(z_p, Wd1, bd1r, relu=True)
    xh = _mm_bias(dd, Wd2, bd2r, relu=False)
    return z_p[:N], xh[:N], zg
